# baseline (speedup 1.0000x reference)
import functools

import jax
import jax.numpy as jnp
from jax import lax
from jax.experimental import pallas as pl
from jax.experimental.pallas import tpu as pltpu
from jax.experimental.pallas import tpu_sc as plsc

N = 10000
E = 320000
D = 128
ONES = 16
DE = D + ONES
NPAD = 10240
CHUNK = 40
WCH = 128
NC, NS = 2, 16
NW = NC * NS
SC_ROWS_PER_TILE = NPAD // NS
EPT_CHUNKS = -(-E // (NW * CHUNK))
EPT = EPT_CHUNKS * CHUNK
E_PAD = EPT * NW

MM_BLK = 1024
CB_BLK = 1000


def _vext_body(x_ref, w_ref, b_ref, o_ref):
    mm = lax.dot_general(x_ref[...], w_ref[...], (((1,), (1,)), ((), ())),
                         preferred_element_type=jnp.float32)
    mm = mm + b_ref[0, :][None, :]
    rid = pl.program_id(0) * MM_BLK + lax.broadcasted_iota(jnp.int32, (MM_BLK, 1), 0)
    valid = rid < N
    feat = jnp.where(valid, mm, 0.0)
    ones = jnp.where(jnp.broadcast_to(valid, (MM_BLK, ONES)), 1.0, 0.0)
    o_ref[...] = jnp.concatenate([feat, ones], axis=1)


_vext_call = pl.pallas_call(
    _vext_body,
    grid=(NPAD // MM_BLK,),
    in_specs=[
        pl.BlockSpec((MM_BLK, D), lambda i: (i, 0)),
        pl.BlockSpec((D, D), lambda i: (0, 0)),
        pl.BlockSpec((8, D), lambda i: (0, 0)),
    ],
    out_specs=pl.BlockSpec((MM_BLK, DE), lambda i: (i, 0)),
    out_shape=jax.ShapeDtypeStruct((NPAD, DE), jnp.float32),
)


NBUF = 6
LOOKAHEAD = 3


def _sc_body(v_hbm, row_hbm, col_hbm, out_hbm,
             idxr0, idxr1, idxr2, idxr3, idxr4, idxr5,
             idxc0, idxc1, idxc2, idxc3, idxc4, idxc5,
             rows0, rows1, rows2, rows3, rows4, rows5, agg,
             gsem0, gsem1, gsem2, gsem3, gsem4, gsem5,
             ssem0, ssem1, ssem2, ssem3, ssem4, ssem5):
    idxr = (idxr0, idxr1, idxr2, idxr3, idxr4, idxr5)
    idxc = (idxc0, idxc1, idxc2, idxc3, idxc4, idxc5)
    rows = (rows0, rows1, rows2, rows3, rows4, rows5)
    gsem = (gsem0, gsem1, gsem2, gsem3, gsem4, gsem5)
    ssem = (ssem0, ssem1, ssem2, ssem3, ssem4, ssem5)
    c = lax.axis_index("c")
    s = lax.axis_index("s")
    wid = s * NC + c
    stripe = s * SC_ROWS_PER_TILE

    for kk in range(SC_ROWS_PER_TILE // WCH):
        pltpu.sync_copy(v_hbm.at[pl.ds(NPAD - WCH, WCH), :],
                        agg.at[pl.ds(stripe + kk * WCH, WCH), :])
    plsc.subcore_barrier()

    base = wid * EPT

    for j in range(LOOKAHEAD):
        e0 = pl.multiple_of(base + j * CHUNK, CHUNK)
        pltpu.sync_copy(row_hbm.at[pl.ds(e0, CHUNK)], idxr[j])
        pltpu.sync_copy(col_hbm.at[pl.ds(e0, CHUNK)], idxc[j])
        pltpu.async_copy(v_hbm.at[idxr[j]], rows[j], gsem[j])

    def stage(k, b):
        b2 = (b + LOOKAHEAD) % NBUF

        @pl.when(k < EPT_CHUNKS)
        def _():
            pltpu.make_async_copy(v_hbm.at[idxr[b]], rows[b], gsem[b]).wait()

            @pl.when(k + LOOKAHEAD < EPT_CHUNKS)
            def _():
                @pl.when(k + LOOKAHEAD >= NBUF)
                def _():
                    pltpu.make_async_copy(
                        rows[b2], agg.at[idxc[b2]], ssem[b2]).wait()
                e1 = pl.multiple_of(base + (k + LOOKAHEAD) * CHUNK, CHUNK)
                pltpu.sync_copy(row_hbm.at[pl.ds(e1, CHUNK)], idxr[b2])
                pltpu.sync_copy(col_hbm.at[pl.ds(e1, CHUNK)], idxc[b2])
                pltpu.async_copy(v_hbm.at[idxr[b2]], rows[b2], gsem[b2])

            pltpu.async_copy(rows[b], agg.at[idxc[b]], ssem[b], add=True)

    def outer(i, carry):
        for b in range(NBUF):
            stage(NBUF * i + b, b)
        return carry

    lax.fori_loop(0, (EPT_CHUNKS + NBUF - 1) // NBUF, outer, 0)
    for k in range(max(EPT_CHUNKS - NBUF, 0), EPT_CHUNKS):
        b = k % NBUF
        pltpu.make_async_copy(rows[b], agg.at[idxc[b]], ssem[b]).wait()
    plsc.subcore_barrier()

    pltpu.sync_copy(agg.at[pl.ds(stripe, SC_ROWS_PER_TILE), :],
                    out_hbm.at[c, pl.ds(stripe, SC_ROWS_PER_TILE), :])


@functools.cache
def _sc_call():
    return pl.kernel(
        _sc_body,
        out_type=jax.ShapeDtypeStruct((NC, NPAD, DE), jnp.float32),
        mesh=plsc.VectorSubcoreMesh(core_axis_name="c", subcore_axis_name="s",
                                    num_cores=NC, num_subcores=NS),
        compiler_params=pltpu.CompilerParams(use_tc_tiling_on_sc=False),
        scratch_types=(
            [pltpu.VMEM((CHUNK,), jnp.int32)] * (2 * NBUF)
            + [pltpu.VMEM((CHUNK, DE), jnp.float32)] * NBUF
            + [pltpu.VMEM_SHARED((NPAD, DE), jnp.float32)]
            + [pltpu.SemaphoreType.DMA] * (2 * NBUF)
        ),
    )


def _combine_body(a_ref, b_ref, o_ref):
    sacc = a_ref[0] + b_ref[0]
    deg = sacc[:, D:D + 1]
    o_ref[...] = sacc[:, :D] / jnp.maximum(deg, 1.0)


_combine_call = pl.pallas_call(
    _combine_body,
    grid=(N // CB_BLK,),
    in_specs=[
        pl.BlockSpec((1, CB_BLK, DE), lambda i: (0, i, 0)),
        pl.BlockSpec((1, CB_BLK, DE), lambda i: (1, i, 0)),
    ],
    out_specs=pl.BlockSpec((CB_BLK, D), lambda i: (i, 0)),
    out_shape=jax.ShapeDtypeStruct((N, D), jnp.float32),
)


def kernel(x, edge_index, Wq, bq, Wk, bk, Wv, bv, attn):
    row = edge_index[0]
    col = edge_index[1]
    pad = jnp.full((E_PAD - E,), N, jnp.int32)
    row_p = jnp.concatenate([row, pad])
    col_p = jnp.concatenate([col, pad])
    x_p = jnp.pad(x, ((0, NPAD - N), (0, 0)))
    bv2 = jnp.broadcast_to(bv[None, :], (8, D))

    v_ext = _vext_call(x_p, Wv, bv2)
    partials = _sc_call()(v_ext, row_p, col_p)
    return _combine_call(partials, partials)


# speedup vs baseline: 1.0801x; 1.0801x over previous
import functools

import jax
import jax.numpy as jnp
from jax import lax
from jax.experimental import pallas as pl
from jax.experimental.pallas import tpu as pltpu
from jax.experimental.pallas import tpu_sc as plsc

N = 10000
E = 320000
D = 128
ONES = 16
DE = D + ONES
NPAD = 10240
CHUNK = 64
WCH = 128
CPT_A = 133
CPT_B = 180
CPT_MAX = max(CPT_A, CPT_B)
NC, NS = 2, 16
NW = NC * NS
SC_ROWS_PER_TILE = NPAD // NS
E_PAD = NS * (CPT_A + CPT_B) * CHUNK
assert E_PAD >= E

MM_BLK = 1024
CB_BLK = 1000


def _vext_body(x_ref, w_ref, b_ref, o_ref):
    mm = lax.dot_general(x_ref[...], w_ref[...], (((1,), (1,)), ((), ())),
                         preferred_element_type=jnp.float32)
    mm = mm + b_ref[0, :][None, :]
    rid = pl.program_id(0) * MM_BLK + lax.broadcasted_iota(jnp.int32, (MM_BLK, 1), 0)
    valid = rid < N
    feat = jnp.where(valid, mm, 0.0)
    ones = jnp.where(jnp.broadcast_to(valid, (MM_BLK, ONES)), 1.0, 0.0)
    o_ref[...] = jnp.concatenate([feat, ones], axis=1)


_vext_call = pl.pallas_call(
    _vext_body,
    grid=(NPAD // MM_BLK,),
    in_specs=[
        pl.BlockSpec((MM_BLK, D), lambda i: (i, 0)),
        pl.BlockSpec((D, D), lambda i: (0, 0)),
        pl.BlockSpec((8, D), lambda i: (0, 0)),
    ],
    out_specs=pl.BlockSpec((MM_BLK, DE), lambda i: (i, 0)),
    out_shape=jax.ShapeDtypeStruct((NPAD, DE), jnp.float32),
)


NBUF = 4
LOOKAHEAD = 2


def _sc_body(v_hbm, row_hbm, col_hbm, out_hbm,
             idxr0, idxr1, idxr2, idxr3, idxc0, idxc1, idxc2, idxc3,
             rows0, rows1, rows2, rows3, agg,
             gsem0, gsem1, gsem2, gsem3, ssem0, ssem1, ssem2, ssem3):
    idxr = (idxr0, idxr1, idxr2, idxr3)
    idxc = (idxc0, idxc1, idxc2, idxc3)
    rows = (rows0, rows1, rows2, rows3)
    gsem = (gsem0, gsem1, gsem2, gsem3)
    ssem = (ssem0, ssem1, ssem2, ssem3)
    c = lax.axis_index("c")
    s = lax.axis_index("s")
    stripe = s * SC_ROWS_PER_TILE

    for kk in range(SC_ROWS_PER_TILE // WCH):
        pltpu.sync_copy(v_hbm.at[pl.ds(NPAD - WCH, WCH), :],
                        agg.at[pl.ds(stripe + kk * WCH, WCH), :])
    plsc.subcore_barrier()

    nch = lax.select(c == 0, CPT_A, CPT_B)
    base = lax.select(c == 0, s * (CPT_A * CHUNK),
                      NS * (CPT_A * CHUNK) + s * (CPT_B * CHUNK))

    for j in range(LOOKAHEAD):
        e0 = pl.multiple_of(base + j * CHUNK, CHUNK)
        pltpu.sync_copy(row_hbm.at[pl.ds(e0, CHUNK)], idxr[j])
        pltpu.sync_copy(col_hbm.at[pl.ds(e0, CHUNK)], idxc[j])
        pltpu.async_copy(v_hbm.at[idxr[j]], rows[j], gsem[j])

    def stage(k, b):
        b2 = (b + LOOKAHEAD) % NBUF

        @pl.when(k < nch)
        def _():
            pltpu.make_async_copy(v_hbm.at[idxr[b]], rows[b], gsem[b]).wait()

            @pl.when(k + LOOKAHEAD < nch)
            def _():
                @pl.when(k + LOOKAHEAD >= NBUF)
                def _():
                    pltpu.make_async_copy(
                        rows[b2], agg.at[idxc[b2]], ssem[b2]).wait()
                e1 = pl.multiple_of(base + (k + LOOKAHEAD) * CHUNK, CHUNK)
                pltpu.sync_copy(row_hbm.at[pl.ds(e1, CHUNK)], idxr[b2])
                pltpu.sync_copy(col_hbm.at[pl.ds(e1, CHUNK)], idxc[b2])
                pltpu.async_copy(v_hbm.at[idxr[b2]], rows[b2], gsem[b2])

            pltpu.async_copy(rows[b], agg.at[idxc[b]], ssem[b], add=True)

    def outer(i, carry):
        for b in range(NBUF):
            stage(NBUF * i + b, b)
        return carry

    lax.fori_loop(0, (CPT_MAX + NBUF - 1) // NBUF, outer, 0)
    for b in range(NBUF):
        pltpu.make_async_copy(rows[b], agg.at[idxc[b]], ssem[b]).wait()
    plsc.subcore_barrier()

    pltpu.sync_copy(agg.at[pl.ds(stripe, SC_ROWS_PER_TILE), :],
                    out_hbm.at[c, pl.ds(stripe, SC_ROWS_PER_TILE), :])


@functools.cache
def _sc_call():
    return pl.kernel(
        _sc_body,
        out_type=jax.ShapeDtypeStruct((NC, NPAD, DE), jnp.float32),
        mesh=plsc.VectorSubcoreMesh(core_axis_name="c", subcore_axis_name="s",
                                    num_cores=NC, num_subcores=NS),
        compiler_params=pltpu.CompilerParams(use_tc_tiling_on_sc=False),
        scratch_types=(
            [pltpu.VMEM((CHUNK,), jnp.int32)] * (2 * NBUF)
            + [pltpu.VMEM((CHUNK, DE), jnp.float32)] * NBUF
            + [pltpu.VMEM_SHARED((NPAD, DE), jnp.float32)]
            + [pltpu.SemaphoreType.DMA] * (2 * NBUF)
        ),
    )


def _combine_body(a_ref, b_ref, o_ref):
    sacc = a_ref[0] + b_ref[0]
    deg = sacc[:, D:D + 1]
    o_ref[...] = sacc[:, :D] / jnp.maximum(deg, 1.0)


_combine_call = pl.pallas_call(
    _combine_body,
    grid=(N // CB_BLK,),
    in_specs=[
        pl.BlockSpec((1, CB_BLK, DE), lambda i: (0, i, 0)),
        pl.BlockSpec((1, CB_BLK, DE), lambda i: (1, i, 0)),
    ],
    out_specs=pl.BlockSpec((CB_BLK, D), lambda i: (i, 0)),
    out_shape=jax.ShapeDtypeStruct((N, D), jnp.float32),
)


def kernel(x, edge_index, Wq, bq, Wk, bk, Wv, bv, attn):
    row = edge_index[0]
    col = edge_index[1]
    pad = jnp.full((E_PAD - E,), N, jnp.int32)
    row_p = jnp.concatenate([row, pad])
    col_p = jnp.concatenate([col, pad])
    x_p = jnp.pad(x, ((0, NPAD - N), (0, 0)))
    bv2 = jnp.broadcast_to(bv[None, :], (8, D))

    v_ext = _vext_call(x_p, Wv, bv2)
    partials = _sc_call()(v_ext, row_p, col_p)
    return _combine_call(partials, partials)
